# baseline (speedup 1.0000x reference)
import functools

import numpy as np
import jax
import jax.numpy as jnp
from jax import lax
from jax.experimental import pallas as pl
from jax.experimental.pallas import tpu as pltpu
from jax.experimental.pallas import tpu_sc as plsc

N = 10000
D = 128
E = 320000

BE = 2000

def _prep_body(x_ref, rw_ref, sw2_ref, rf_ref, pa_ref):
    x = x_ref[...]
    z = lax.dot_general(x, rw_ref[...], (((1,), (1,)), ((), ())),
                        preferred_element_type=jnp.float32)
    sc2 = lax.dot_general(x, sw2_ref[...], (((1,), (1,)), ((), ())),
                          preferred_element_type=jnp.float32)
    rf_ref[...] = (z * jax.nn.sigmoid(sc2[:, 1:2])).astype(jnp.bfloat16)
    pa_ref[...] = jax.nn.sigmoid(sc2[:, 0:1])


def _edge_prep(review_feat, review_w, prob_score_w, review_score_w):
    sw2 = jnp.concatenate([prob_score_w, review_score_w], axis=0)
    grid = (E // BE,)
    return pl.pallas_call(
        _prep_body,
        grid=grid,
        in_specs=[
            pl.BlockSpec((BE, D), lambda i: (i, 0)),
            pl.BlockSpec((D, D), lambda i: (0, 0)),
            pl.BlockSpec((2, D), lambda i: (0, 0)),
        ],
        out_specs=[
            pl.BlockSpec((BE, D), lambda i: (i, 0)),
            pl.BlockSpec((BE, 1), lambda i: (i, 0)),
        ],
        out_shape=[
            jax.ShapeDtypeStruct((E, D), jnp.bfloat16),
            jax.ShapeDtypeStruct((E, 1), jnp.float32),
        ],
    )(review_feat, review_w, sw2)


_PERM_F = np.empty(D, dtype=np.int64)
for _p in range(D):
    _q, _r = _p // 32, _p % 32
    _PERM_F[_p] = 32 * _q + 2 * _r if _r < 16 else 32 * _q + 2 * (_r - 16) + 1
_PW = np.zeros((D, D), dtype=np.float32)
for _p in range(D):
    _PW[_PERM_F[_p], _p] = 1.0
_PU = np.zeros((D, D), dtype=np.float32)
for _p in range(D):
    _PU[_p, _PERM_F[_p]] = 1.0

BT = 1000

def _wperm_body(w_ref, pw_ref, o_ref):
    o_ref[...] = lax.dot_general(w_ref[...], pw_ref[...],
                                 (((1,), (0,)), ((), ())),
                                 preferred_element_type=jnp.float32)


def _wperm(weight):
    grid = (N // BT,)
    return pl.pallas_call(
        _wperm_body,
        grid=grid,
        in_specs=[
            pl.BlockSpec((BT, D), lambda i: (i, 0)),
            pl.BlockSpec((D, D), lambda i: (0, 0)),
        ],
        out_specs=pl.BlockSpec((BT, D), lambda i: (i, 0)),
        out_shape=jax.ShapeDtypeStruct((N, D), jnp.float32),
    )(weight, jnp.asarray(_PW))


GSZ = 2000
NG = E // GSZ
NTILES = 32
GPT = NG // NTILES

_sc_mesh = plsc.VectorSubcoreMesh(core_axis_name="c", subcore_axis_name="s")


@functools.partial(
    pl.kernel,
    out_type=[jax.ShapeDtypeStruct((NG, 1, GSZ), jnp.float32),
              jax.ShapeDtypeStruct((NG, 1, GSZ), jnp.float32)],
    mesh=_sc_mesh,
    compiler_params=pltpu.CompilerParams(needs_layout_passes=False),
    scratch_types=[
        pltpu.VMEM((N,), jnp.float32),
        pltpu.VMEM((1, GSZ), jnp.int32),
        pltpu.VMEM((1, GSZ), jnp.float32),
        pltpu.VMEM((1, GSZ), jnp.float32),
        pltpu.VMEM((1, GSZ), jnp.float32),
    ],
)
def _sc_scalars(src_hbm, pa_hbm, cj_hbm, s1_hbm, s2_hbm,
                cj_v, src_v, pa_v, s1_v, s2_v):
    c = lax.axis_index("c")
    s = lax.axis_index("s")
    wid = s * 2 + c
    pltpu.sync_copy(cj_hbm, cj_v)

    def _group(gi, carry):
        g = wid * GPT + gi
        pltpu.sync_copy(src_hbm.at[g], src_v)
        pltpu.sync_copy(pa_hbm.at[g], pa_v)
        for i2 in range(GSZ // 16):
            sl = pl.ds(i2 * 16, 16)
            cj16 = plsc.load_gather(cj_v, [src_v[0, sl]])
            s1_v[0, sl] = pa_v[0, sl] * cj16
            s2_v[0, sl] = cj16
        pltpu.sync_copy(s1_v, s1_hbm.at[g])
        pltpu.sync_copy(s2_v, s2_hbm.at[g])
        return carry
    lax.fori_loop(0, GPT, _group, 0)


CH = 80
SUP = GSZ // CH
ZCH = 80
NZ = N // ZCH
ZBASE = NZ // 16
ZEXTRA = NZ - ZBASE * 16


@functools.partial(
    pl.kernel,
    out_type=jax.ShapeDtypeStruct((2, N, D), jnp.float32),
    mesh=_sc_mesh,
    compiler_params=pltpu.CompilerParams(needs_layout_passes=False),
    scratch_types=[
        pltpu.VMEM((1, GSZ), jnp.int32),
        pltpu.VMEM((1, GSZ), jnp.int32),
        pltpu.VMEM((1, GSZ), jnp.float32),
        pltpu.VMEM((1, GSZ), jnp.float32),
        pltpu.VMEM((CH,), jnp.int32),
        pltpu.VMEM((CH,), jnp.int32),
        pltpu.VMEM((CH,), jnp.int32),
        pltpu.VMEM((CH,), jnp.int32),
        pltpu.VMEM((CH, D), jnp.float32),
        pltpu.VMEM((CH, D), jnp.float32),
        pltpu.VMEM((CH, D), jnp.bfloat16),
        pltpu.VMEM((CH, D), jnp.bfloat16),
        pltpu.VMEM_SHARED((N, D), jnp.float32),
        pltpu.SemaphoreType.DMA,
        pltpu.SemaphoreType.DMA,
        pltpu.SemaphoreType.DMA,
        pltpu.SemaphoreType.DMA,
    ],
)
def _sc_main(src_hbm, dst_hbm, s1_hbm, s2_hbm, rf_hbm, w_hbm, out_hbm,
             src_v, dst_v, s1_v, s2_v, gia, gib, sia, sib,
             ta, tb, rfa, rfb, acc, semia, semib, semsa, semsb):
    c = lax.axis_index("c")
    s = lax.axis_index("s")
    wid = s * 2 + c
    z16 = jnp.zeros((16,), jnp.int32)

    def _zrow(r, carry):
        for j in range(8):
            ta[r, pl.ds(j * 16, 16)] = jnp.zeros((16,), jnp.float32)
        return carry
    lax.fori_loop(0, ZCH, _zrow, 0)
    nz = ZBASE + jnp.where(s < ZEXTRA, 1, 0)

    def _zero_chunk(k, carry):
        blk = k * 16 + s
        pltpu.sync_copy(ta.at[pl.ds(0, ZCH)], acc.at[pl.ds(blk * ZCH, ZCH)])
        return carry
    lax.fori_loop(0, nz, _zero_chunk, 0)
    plsc.subcore_barrier()

    def issue(k, base_e, gi_ref, t_ref, rf_ref, semi):
        for gg in range(CH // 16):
            sl = pl.ds(gg * 16, 16)
            gi_ref[sl] = src_v[0, pl.ds(k * CH + gg * 16, 16)]
        pltpu.async_copy(rf_hbm.at[pl.ds(base_e + k * CH, CH)], rf_ref, semi)
        pltpu.async_copy(w_hbm.at[gi_ref], t_ref, semi)

    def drain_in(gi_ref, t_ref, rf_ref, semi):
        pltpu.make_async_copy(rf_hbm.at[pl.ds(0, CH)], rf_ref, semi).wait()
        pltpu.make_async_copy(w_hbm.at[gi_ref], t_ref, semi).wait()

    def compute(k, si_ref, t_ref, rf_ref):
        for gg in range(CH // 16):
            sl = pl.ds(gg * 16, 16)
            si_ref[sl] = dst_v[0, pl.ds(k * CH + gg * 16, 16)]

        def _edge(e, carry2):
            idx = jnp.full((16,), k * CH + e, jnp.int32)
            s1 = plsc.load_gather(s1_v, [z16, idx])
            s2 = plsc.load_gather(s2_v, [z16, idx])
            for q in range(4):
                r32 = rf_ref[e, pl.ds(32 * q, 32)]
                ra, rb = plsc.unpack(r32, format=plsc.PackFormat.INTERLEAVED)
                sa = pl.ds(32 * q, 16)
                sb = pl.ds(32 * q + 16, 16)
                t_ref[e, sa] = t_ref[e, sa] * s1 + ra * s2
                t_ref[e, sb] = t_ref[e, sb] * s1 + rb * s2
            return carry2
        lax.fori_loop(0, CH, _edge, 0)

    def issue_scatter(si_ref, t_ref, sems):
        pltpu.async_copy(t_ref, acc.at[si_ref], sems, add=True)

    def drain_scatter(si_ref, t_ref, sems):
        pltpu.make_async_copy(t_ref, acc.at[si_ref], sems).wait()

    def _super(sp, carry):
        g = wid * GPT + sp
        base_e = g * GSZ
        pltpu.sync_copy(src_hbm.at[g], src_v)
        pltpu.sync_copy(dst_hbm.at[g], dst_v)
        pltpu.sync_copy(s1_hbm.at[g], s1_v)
        pltpu.sync_copy(s2_hbm.at[g], s2_v)

        issue(0, base_e, gia, ta, rfa, semia)
        issue(1, base_e, gib, tb, rfb, semib)

        def _pair(j, carry2):
            a = 2 * j
            drain_in(gia, ta, rfa, semia)
            compute(a, sia, ta, rfa)
            issue_scatter(sia, ta, semsa)
            drain_in(gib, tb, rfb, semib)
            compute(a + 1, sib, tb, rfb)
            issue_scatter(sib, tb, semsb)
            drain_scatter(sia, ta, semsa)
            issue(a + 2, base_e, gia, ta, rfa, semia)
            drain_scatter(sib, tb, semsb)

            @pl.when(a + 3 < SUP)
            def _():
                issue(a + 3, base_e, gib, tb, rfb, semib)
            return carry2
        lax.fori_loop(0, (SUP - 1) // 2, _pair, 0)

        drain_in(gia, ta, rfa, semia)
        compute(SUP - 1, sia, ta, rfa)
        issue_scatter(sia, ta, semsa)
        drain_scatter(sia, ta, semsa)
        return carry
    lax.fori_loop(0, GPT, _super, 0)

    plsc.subcore_barrier()

    def _out_chunk(k, carry):
        blk = k * 16 + s
        pltpu.sync_copy(acc.at[pl.ds(blk * ZCH, ZCH)],
                        out_hbm.at[c, pl.ds(blk * ZCH, ZCH)])
        return carry
    lax.fori_loop(0, nz, _out_chunk, 0)


BN = 1000

def _combine_body(p_ref, pu_ref, ci_ref, o_ref):
    p = p_ref[...]
    su = p[0] + p[1]
    o_ref[...] = lax.dot_general(su, pu_ref[...], (((1,), (0,)), ((), ())),
                                 preferred_element_type=jnp.float32) * ci_ref[...]


def _combine(partials, ci):
    grid = (N // BN,)
    return pl.pallas_call(
        _combine_body,
        grid=grid,
        in_specs=[
            pl.BlockSpec((2, BN, D), lambda i: (0, i, 0)),
            pl.BlockSpec((D, D), lambda i: (0, 0)),
            pl.BlockSpec((BN, 1), lambda i: (i, 0)),
        ],
        out_specs=pl.BlockSpec((BN, D), lambda i: (i, 0)),
        out_shape=jax.ShapeDtypeStruct((N, D), jnp.float32),
    )(partials, jnp.asarray(_PU), ci)


def kernel(edge_index, review_feat, cj, ci, weight, prob_score_w,
           review_score_w, review_w):
    src = edge_index[0].astype(jnp.int32).reshape(NG, 1, GSZ)
    dst = edge_index[1].astype(jnp.int32).reshape(NG, 1, GSZ)
    rf, pa = _edge_prep(review_feat, review_w, prob_score_w, review_score_w)
    wp = _wperm(weight)
    s1, s2 = _sc_scalars(src, pa.reshape(NG, 1, GSZ), cj.reshape(N))
    partials = _sc_main(src, dst, s1, s2, rf, wp)
    return _combine(partials, ci)


# speedup vs baseline: 1.2024x; 1.2024x over previous
import functools

import jax
import jax.numpy as jnp
from jax import lax
from jax.experimental import pallas as pl
from jax.experimental.pallas import tpu as pltpu
from jax.experimental.pallas import tpu_sc as plsc

N = 10000
D = 128
E = 320000

BE = 2000

def _prep_body(x_ref, rw_ref, sw2_ref, rf_ref, pa_ref):
    x = x_ref[...]
    z = lax.dot_general(x, rw_ref[...], (((1,), (1,)), ((), ())),
                        preferred_element_type=jnp.float32)
    sc2 = lax.dot_general(x, sw2_ref[...], (((1,), (1,)), ((), ())),
                          preferred_element_type=jnp.float32)
    rf_ref[...] = z * jax.nn.sigmoid(sc2[:, 1:2])
    pa_ref[...] = jax.nn.sigmoid(sc2[:, 0:1])


def _edge_prep(review_feat, review_w, prob_score_w, review_score_w):
    sw2 = jnp.concatenate([prob_score_w, review_score_w], axis=0)
    grid = (E // BE,)
    return pl.pallas_call(
        _prep_body,
        grid=grid,
        in_specs=[
            pl.BlockSpec((BE, D), lambda i: (i, 0)),
            pl.BlockSpec((D, D), lambda i: (0, 0)),
            pl.BlockSpec((2, D), lambda i: (0, 0)),
        ],
        out_specs=[
            pl.BlockSpec((BE, D), lambda i: (i, 0)),
            pl.BlockSpec((BE, 1), lambda i: (i, 0)),
        ],
        out_shape=[
            jax.ShapeDtypeStruct((E, D), jnp.float32),
            jax.ShapeDtypeStruct((E, 1), jnp.float32),
        ],
    )(review_feat, review_w, sw2)


GSZ = 2000
NG = E // GSZ
NTILES = 32
GPT = NG // NTILES

_sc_mesh = plsc.VectorSubcoreMesh(core_axis_name="c", subcore_axis_name="s")


@functools.partial(
    pl.kernel,
    out_type=[jax.ShapeDtypeStruct((NG, 1, GSZ), jnp.float32),
              jax.ShapeDtypeStruct((NG, 1, GSZ), jnp.float32)],
    mesh=_sc_mesh,
    compiler_params=pltpu.CompilerParams(needs_layout_passes=False),
    scratch_types=[
        pltpu.VMEM((N,), jnp.float32),
        pltpu.VMEM((1, GSZ), jnp.int32),
        pltpu.VMEM((1, GSZ), jnp.float32),
        pltpu.VMEM((1, GSZ), jnp.float32),
        pltpu.VMEM((1, GSZ), jnp.float32),
    ],
)
def _sc_scalars(src_hbm, pa_hbm, cj_hbm, s1_hbm, s2_hbm,
                cj_v, src_v, pa_v, s1_v, s2_v):
    c = lax.axis_index("c")
    s = lax.axis_index("s")
    wid = s * 2 + c
    pltpu.sync_copy(cj_hbm, cj_v)

    def _group(gi, carry):
        g = wid * GPT + gi
        pltpu.sync_copy(src_hbm.at[g], src_v)
        pltpu.sync_copy(pa_hbm.at[g], pa_v)
        for i2 in range(GSZ // 16):
            sl = pl.ds(i2 * 16, 16)
            cj16 = plsc.load_gather(cj_v, [src_v[0, sl]])
            s1_v[0, sl] = pa_v[0, sl] * cj16
            s2_v[0, sl] = cj16
        pltpu.sync_copy(s1_v, s1_hbm.at[g])
        pltpu.sync_copy(s2_v, s2_hbm.at[g])
        return carry
    lax.fori_loop(0, GPT, _group, 0)


CH = 80
SUP = GSZ // CH
ZCH = 80
NZ = N // ZCH
ZBASE = NZ // 16
ZEXTRA = NZ - ZBASE * 16


@functools.partial(
    pl.kernel,
    out_type=jax.ShapeDtypeStruct((2, N, D), jnp.float32),
    mesh=_sc_mesh,
    compiler_params=pltpu.CompilerParams(needs_layout_passes=False),
    scratch_types=[
        pltpu.VMEM((1, GSZ), jnp.int32),
        pltpu.VMEM((1, GSZ), jnp.int32),
        pltpu.VMEM((1, GSZ), jnp.float32),
        pltpu.VMEM((1, GSZ), jnp.float32),
        pltpu.VMEM((CH,), jnp.int32),
        pltpu.VMEM((CH,), jnp.int32),
        pltpu.VMEM((CH,), jnp.int32),
        pltpu.VMEM((CH,), jnp.int32),
        pltpu.VMEM((CH, D), jnp.float32),
        pltpu.VMEM((CH, D), jnp.float32),
        pltpu.VMEM((CH, D), jnp.float32),
        pltpu.VMEM((CH, D), jnp.float32),
        pltpu.VMEM_SHARED((N, D), jnp.float32),
        pltpu.SemaphoreType.DMA,
        pltpu.SemaphoreType.DMA,
        pltpu.SemaphoreType.DMA,
        pltpu.SemaphoreType.DMA,
    ],
)
def _sc_main(src_hbm, dst_hbm, s1_hbm, s2_hbm, rf_hbm, w_hbm, out_hbm,
             src_v, dst_v, s1_v, s2_v, gia, gib, sia, sib,
             ta, tb, rfa, rfb, acc, semia, semib, semsa, semsb):
    c = lax.axis_index("c")
    s = lax.axis_index("s")
    wid = s * 2 + c
    z16 = jnp.zeros((16,), jnp.int32)

    def _zrow(r, carry):
        for j in range(8):
            ta[r, pl.ds(j * 16, 16)] = jnp.zeros((16,), jnp.float32)
        return carry
    lax.fori_loop(0, ZCH, _zrow, 0)
    nz = ZBASE + jnp.where(s < ZEXTRA, 1, 0)

    def _zero_chunk(k, carry):
        blk = k * 16 + s
        pltpu.sync_copy(ta.at[pl.ds(0, ZCH)], acc.at[pl.ds(blk * ZCH, ZCH)])
        return carry
    lax.fori_loop(0, nz, _zero_chunk, 0)
    plsc.subcore_barrier()

    def issue(k, base_e, gi_ref, t_ref, rf_ref, semi):
        for gg in range(CH // 16):
            sl = pl.ds(gg * 16, 16)
            gi_ref[sl] = src_v[0, pl.ds(k * CH + gg * 16, 16)]
        pltpu.async_copy(rf_hbm.at[pl.ds(base_e + k * CH, CH)], rf_ref, semi)
        pltpu.async_copy(w_hbm.at[gi_ref], t_ref, semi)

    def drain_in(gi_ref, t_ref, rf_ref, semi):
        pltpu.make_async_copy(rf_hbm.at[pl.ds(0, CH)], rf_ref, semi).wait()
        pltpu.make_async_copy(w_hbm.at[gi_ref], t_ref, semi).wait()

    def compute(k, si_ref, t_ref, rf_ref):
        for gg in range(CH // 16):
            sl = pl.ds(gg * 16, 16)
            si_ref[sl] = dst_v[0, pl.ds(k * CH + gg * 16, 16)]

        def _edge(e, carry2):
            idx = jnp.full((16,), k * CH + e, jnp.int32)
            s1 = plsc.load_gather(s1_v, [z16, idx])
            s2 = plsc.load_gather(s2_v, [z16, idx])
            for j in range(8):
                sj = pl.ds(j * 16, 16)
                t_ref[e, sj] = t_ref[e, sj] * s1 + rf_ref[e, sj] * s2
            return carry2
        lax.fori_loop(0, CH, _edge, 0)

    def issue_scatter(si_ref, t_ref, sems):
        pltpu.async_copy(t_ref, acc.at[si_ref], sems, add=True)

    def drain_scatter(si_ref, t_ref, sems):
        pltpu.make_async_copy(t_ref, acc.at[si_ref], sems).wait()

    def _super(sp, carry):
        g = wid * GPT + sp
        base_e = g * GSZ
        pltpu.sync_copy(src_hbm.at[g], src_v)
        pltpu.sync_copy(dst_hbm.at[g], dst_v)
        pltpu.sync_copy(s1_hbm.at[g], s1_v)
        pltpu.sync_copy(s2_hbm.at[g], s2_v)

        issue(0, base_e, gia, ta, rfa, semia)
        issue(1, base_e, gib, tb, rfb, semib)

        def _pair(j, carry2):
            a = 2 * j
            drain_in(gia, ta, rfa, semia)
            compute(a, sia, ta, rfa)
            issue_scatter(sia, ta, semsa)
            drain_in(gib, tb, rfb, semib)
            compute(a + 1, sib, tb, rfb)
            issue_scatter(sib, tb, semsb)
            drain_scatter(sia, ta, semsa)
            issue(a + 2, base_e, gia, ta, rfa, semia)
            drain_scatter(sib, tb, semsb)

            @pl.when(a + 3 < SUP)
            def _():
                issue(a + 3, base_e, gib, tb, rfb, semib)
            return carry2
        lax.fori_loop(0, (SUP - 1) // 2, _pair, 0)

        drain_in(gia, ta, rfa, semia)
        compute(SUP - 1, sia, ta, rfa)
        issue_scatter(sia, ta, semsa)
        drain_scatter(sia, ta, semsa)
        return carry
    lax.fori_loop(0, GPT, _super, 0)

    plsc.subcore_barrier()

    def _out_chunk(k, carry):
        blk = k * 16 + s
        pltpu.sync_copy(acc.at[pl.ds(blk * ZCH, ZCH)],
                        out_hbm.at[c, pl.ds(blk * ZCH, ZCH)])
        return carry
    lax.fori_loop(0, nz, _out_chunk, 0)


BN = 1000

def _combine_body(p_ref, ci_ref, o_ref):
    p = p_ref[...]
    o_ref[...] = (p[0] + p[1]) * ci_ref[...]


def _combine(partials, ci):
    grid = (N // BN,)
    return pl.pallas_call(
        _combine_body,
        grid=grid,
        in_specs=[
            pl.BlockSpec((2, BN, D), lambda i: (0, i, 0)),
            pl.BlockSpec((BN, 1), lambda i: (i, 0)),
        ],
        out_specs=pl.BlockSpec((BN, D), lambda i: (i, 0)),
        out_shape=jax.ShapeDtypeStruct((N, D), jnp.float32),
    )(partials, ci)


def kernel(edge_index, review_feat, cj, ci, weight, prob_score_w,
           review_score_w, review_w):
    src = edge_index[0].astype(jnp.int32).reshape(NG, 1, GSZ)
    dst = edge_index[1].astype(jnp.int32).reshape(NG, 1, GSZ)
    rf, pa = _edge_prep(review_feat, review_w, prob_score_w, review_score_w)
    s1, s2 = _sc_scalars(src, pa.reshape(NG, 1, GSZ), cj.reshape(N))
    partials = _sc_main(src, dst, s1, s2, rf, weight)
    return _combine(partials, ci)


# speedup vs baseline: 1.2210x; 1.0155x over previous
import functools

import jax
import jax.numpy as jnp
from jax import lax
from jax.experimental import pallas as pl
from jax.experimental.pallas import tpu as pltpu
from jax.experimental.pallas import tpu_sc as plsc

N = 10000
D = 128
E = 320000

BE = 2000

def _prep_body(x_ref, rw_ref, sw2_ref, rf_ref, pa_ref):
    x = x_ref[...]
    z = lax.dot_general(x, rw_ref[...], (((1,), (1,)), ((), ())),
                        preferred_element_type=jnp.float32)
    sc2 = lax.dot_general(x, sw2_ref[...], (((1,), (1,)), ((), ())),
                          preferred_element_type=jnp.float32)
    rf_ref[...] = z * jax.nn.sigmoid(sc2[:, 1:2])
    pa_ref[...] = jax.nn.sigmoid(sc2[:, 0:1])


def _edge_prep(review_feat, review_w, prob_score_w, review_score_w):
    sw2 = jnp.concatenate([prob_score_w, review_score_w], axis=0)
    grid = (E // BE,)
    return pl.pallas_call(
        _prep_body,
        grid=grid,
        in_specs=[
            pl.BlockSpec((BE, D), lambda i: (i, 0)),
            pl.BlockSpec((D, D), lambda i: (0, 0)),
            pl.BlockSpec((2, D), lambda i: (0, 0)),
        ],
        out_specs=[
            pl.BlockSpec((BE, D), lambda i: (i, 0)),
            pl.BlockSpec((BE, 1), lambda i: (i, 0)),
        ],
        out_shape=[
            jax.ShapeDtypeStruct((E, D), jnp.float32),
            jax.ShapeDtypeStruct((E, 1), jnp.float32),
        ],
    )(review_feat, review_w, sw2)


GSZ = 2000
NG = E // GSZ
NTILES = 32
GPT = NG // NTILES

_sc_mesh = plsc.VectorSubcoreMesh(core_axis_name="c", subcore_axis_name="s")


@functools.partial(
    pl.kernel,
    out_type=[jax.ShapeDtypeStruct((NG, 1, GSZ), jnp.float32),
              jax.ShapeDtypeStruct((NG, 1, GSZ), jnp.float32)],
    mesh=_sc_mesh,
    compiler_params=pltpu.CompilerParams(needs_layout_passes=False),
    scratch_types=[
        pltpu.VMEM((N,), jnp.float32),
        pltpu.VMEM((1, GSZ), jnp.int32),
        pltpu.VMEM((1, GSZ), jnp.float32),
        pltpu.VMEM((1, GSZ), jnp.float32),
        pltpu.VMEM((1, GSZ), jnp.float32),
    ],
)
def _sc_scalars(src_hbm, pa_hbm, cj_hbm, s1_hbm, s2_hbm,
                cj_v, src_v, pa_v, s1_v, s2_v):
    c = lax.axis_index("c")
    s = lax.axis_index("s")
    wid = s * 2 + c
    pltpu.sync_copy(cj_hbm, cj_v)

    def _group(gi, carry):
        g = wid * GPT + gi
        pltpu.sync_copy(src_hbm.at[g], src_v)
        pltpu.sync_copy(pa_hbm.at[g], pa_v)
        for i2 in range(GSZ // 16):
            sl = pl.ds(i2 * 16, 16)
            cj16 = plsc.load_gather(cj_v, [src_v[0, sl]])
            s1_v[0, sl] = pa_v[0, sl] * cj16
            s2_v[0, sl] = cj16
        pltpu.sync_copy(s1_v, s1_hbm.at[g])
        pltpu.sync_copy(s2_v, s2_hbm.at[g])
        return carry
    lax.fori_loop(0, GPT, _group, 0)


CH = 80
SUP = GSZ // CH
ZCH = 80
NZ = N // ZCH
ZBASE = NZ // 16
ZEXTRA = NZ - ZBASE * 16


@functools.partial(
    pl.kernel,
    out_type=jax.ShapeDtypeStruct((2, N, D), jnp.float32),
    mesh=_sc_mesh,
    compiler_params=pltpu.CompilerParams(needs_layout_passes=False),
    scratch_types=[
        pltpu.VMEM((1, GSZ), jnp.int32),
        pltpu.VMEM((1, GSZ), jnp.int32),
        pltpu.VMEM((1, GSZ), jnp.float32),
        pltpu.VMEM((1, GSZ), jnp.float32),
        pltpu.VMEM((CH,), jnp.int32),
        pltpu.VMEM((CH,), jnp.int32),
        pltpu.VMEM((CH,), jnp.int32),
        pltpu.VMEM((CH,), jnp.int32),
        pltpu.VMEM((CH, D), jnp.float32),
        pltpu.VMEM((CH, D), jnp.float32),
        pltpu.VMEM((CH, D), jnp.float32),
        pltpu.VMEM((CH, D), jnp.float32),
        pltpu.VMEM_SHARED((N, D), jnp.float32),
        pltpu.SemaphoreType.DMA,
        pltpu.SemaphoreType.DMA,
        pltpu.SemaphoreType.DMA,
        pltpu.SemaphoreType.DMA,
    ],
)
def _sc_main(src_hbm, dst_hbm, s1_hbm, s2_hbm, rf_hbm, w_hbm, out_hbm,
             src_v, dst_v, s1_v, s2_v, gia, gib, sia, sib,
             ta, tb, rfa, rfb, acc, semia, semib, semsa, semsb):
    c = lax.axis_index("c")
    s = lax.axis_index("s")
    wid = s * 2 + c
    z16 = jnp.zeros((16,), jnp.int32)

    def _zrow(r, carry):
        for j in range(8):
            ta[r, pl.ds(j * 16, 16)] = jnp.zeros((16,), jnp.float32)
        return carry
    lax.fori_loop(0, ZCH, _zrow, 0)
    nz = ZBASE + jnp.where(s < ZEXTRA, 1, 0)

    def _zero_chunk(k, carry):
        blk = k * 16 + s
        pltpu.async_copy(ta.at[pl.ds(0, ZCH)], acc.at[pl.ds(blk * ZCH, ZCH)],
                         semia)
        return carry
    lax.fori_loop(0, nz, _zero_chunk, 0)

    def _zero_drain(k, carry):
        blk = k * 16 + s
        pltpu.make_async_copy(ta.at[pl.ds(0, ZCH)],
                              acc.at[pl.ds(blk * ZCH, ZCH)], semia).wait()
        return carry
    lax.fori_loop(0, nz, _zero_drain, 0)
    plsc.subcore_barrier()

    def issue(k, base_e, gi_ref, t_ref, rf_ref, semi):
        for gg in range(CH // 16):
            sl = pl.ds(gg * 16, 16)
            gi_ref[sl] = src_v[0, pl.ds(k * CH + gg * 16, 16)]
        pltpu.async_copy(rf_hbm.at[pl.ds(base_e + k * CH, CH)], rf_ref, semi)
        pltpu.async_copy(w_hbm.at[gi_ref], t_ref, semi)

    def drain_in(gi_ref, t_ref, rf_ref, semi):
        pltpu.make_async_copy(rf_hbm.at[pl.ds(0, CH)], rf_ref, semi).wait()
        pltpu.make_async_copy(w_hbm.at[gi_ref], t_ref, semi).wait()

    def compute(k, si_ref, t_ref, rf_ref):
        for gg in range(CH // 16):
            sl = pl.ds(gg * 16, 16)
            si_ref[sl] = dst_v[0, pl.ds(k * CH + gg * 16, 16)]

        def _edge(e, carry2):
            idx = jnp.full((16,), k * CH + e, jnp.int32)
            s1 = plsc.load_gather(s1_v, [z16, idx])
            s2 = plsc.load_gather(s2_v, [z16, idx])
            for j in range(8):
                sj = pl.ds(j * 16, 16)
                t_ref[e, sj] = t_ref[e, sj] * s1 + rf_ref[e, sj] * s2
            return carry2
        lax.fori_loop(0, CH, _edge, 0)

    def issue_scatter(si_ref, t_ref, sems):
        pltpu.async_copy(t_ref, acc.at[si_ref], sems, add=True)

    def drain_scatter(si_ref, t_ref, sems):
        pltpu.make_async_copy(t_ref, acc.at[si_ref], sems).wait()

    def _super(sp, carry):
        g = wid * GPT + sp
        base_e = g * GSZ
        pltpu.async_copy(src_hbm.at[g], src_v, semia)
        pltpu.async_copy(dst_hbm.at[g], dst_v, semia)
        pltpu.async_copy(s1_hbm.at[g], s1_v, semia)
        pltpu.async_copy(s2_hbm.at[g], s2_v, semia)
        pltpu.make_async_copy(src_hbm.at[g], src_v, semia).wait()
        pltpu.make_async_copy(dst_hbm.at[g], dst_v, semia).wait()
        pltpu.make_async_copy(s1_hbm.at[g], s1_v, semia).wait()
        pltpu.make_async_copy(s2_hbm.at[g], s2_v, semia).wait()

        issue(0, base_e, gia, ta, rfa, semia)
        issue(1, base_e, gib, tb, rfb, semib)

        def _pair(j, carry2):
            a = 2 * j
            drain_in(gia, ta, rfa, semia)
            compute(a, sia, ta, rfa)
            issue_scatter(sia, ta, semsa)
            drain_in(gib, tb, rfb, semib)
            compute(a + 1, sib, tb, rfb)
            issue_scatter(sib, tb, semsb)
            drain_scatter(sia, ta, semsa)
            issue(a + 2, base_e, gia, ta, rfa, semia)
            drain_scatter(sib, tb, semsb)

            @pl.when(a + 3 < SUP)
            def _():
                issue(a + 3, base_e, gib, tb, rfb, semib)
            return carry2
        lax.fori_loop(0, (SUP - 1) // 2, _pair, 0)

        drain_in(gia, ta, rfa, semia)
        compute(SUP - 1, sia, ta, rfa)
        issue_scatter(sia, ta, semsa)
        drain_scatter(sia, ta, semsa)
        return carry
    lax.fori_loop(0, GPT, _super, 0)

    plsc.subcore_barrier()

    def _out_chunk(k, carry):
        blk = k * 16 + s
        pltpu.async_copy(acc.at[pl.ds(blk * ZCH, ZCH)],
                         out_hbm.at[c, pl.ds(blk * ZCH, ZCH)], semia)
        return carry
    lax.fori_loop(0, nz, _out_chunk, 0)

    def _out_drain(k, carry):
        blk = k * 16 + s
        pltpu.make_async_copy(acc.at[pl.ds(blk * ZCH, ZCH)],
                              out_hbm.at[c, pl.ds(blk * ZCH, ZCH)],
                              semia).wait()
        return carry
    lax.fori_loop(0, nz, _out_drain, 0)


BN = 1000

def _combine_body(p_ref, ci_ref, o_ref):
    p = p_ref[...]
    o_ref[...] = (p[0] + p[1]) * ci_ref[...]


def _combine(partials, ci):
    grid = (N // BN,)
    return pl.pallas_call(
        _combine_body,
        grid=grid,
        in_specs=[
            pl.BlockSpec((2, BN, D), lambda i: (0, i, 0)),
            pl.BlockSpec((BN, 1), lambda i: (i, 0)),
        ],
        out_specs=pl.BlockSpec((BN, D), lambda i: (i, 0)),
        out_shape=jax.ShapeDtypeStruct((N, D), jnp.float32),
    )(partials, ci)


def kernel(edge_index, review_feat, cj, ci, weight, prob_score_w,
           review_score_w, review_w):
    src = edge_index[0].astype(jnp.int32).reshape(NG, 1, GSZ)
    dst = edge_index[1].astype(jnp.int32).reshape(NG, 1, GSZ)
    rf, pa = _edge_prep(review_feat, review_w, prob_score_w, review_score_w)
    s1, s2 = _sc_scalars(src, pa.reshape(NG, 1, GSZ), cj.reshape(N))
    partials = _sc_main(src, dst, s1, s2, rf, weight)
    return _combine(partials, ci)
